# baseline (speedup 1.0000x reference)
import functools

import jax
import jax.numpy as jnp
from jax.experimental import pallas as pl
from jax.experimental.pallas import tpu as pltpu


def _moe_kernel(x_ref, w1_ref, b1_ref, w2_ref, b2_ref, w3_ref, b3_ref,
                wg_ref, bg_ref, out_ref, stats_ref, *, n_task, n_exp):
    xb = x_ref[...]
    tb = xb.shape[0]
    h1dim = w1_ref.shape[1] // n_exp

    h1 = jnp.dot(xb.astype(jnp.bfloat16), w1_ref[...],
                 preferred_element_type=jnp.float32)
    h1 = jnp.maximum(h1 + b1_ref[...], 0.0)

    logits = jnp.dot(xb, wg_ref[...], preferred_element_type=jnp.float32)
    logits = logits + bg_ref[...]

    cols = jax.lax.broadcasted_iota(jnp.int32, (tb, n_exp), 1)

    gates = []
    for t in range(n_task):
        lt = logits[:, t * n_exp:(t + 1) * n_exp]
        m1 = jnp.max(lt, axis=1, keepdims=True)
        i1 = jnp.min(jnp.where(lt == m1, cols, n_exp), axis=1, keepdims=True)
        sel1 = cols == i1
        masked = jnp.where(sel1, -jnp.inf, lt)
        m2 = jnp.max(masked, axis=1, keepdims=True)
        i2 = jnp.min(jnp.where(masked == m2, cols, n_exp), axis=1,
                     keepdims=True)
        sel2 = cols == i2
        z = jnp.exp(m2 - m1)
        g1 = 1.0 / (1.0 + z)
        g2 = z / (1.0 + z)
        gates.append(jnp.where(sel1, g1, 0.0) + jnp.where(sel2, g2, 0.0))

    del h1dim
    h2 = jnp.dot(h1.astype(jnp.bfloat16), w2_ref[...],
                 preferred_element_type=jnp.float32)
    h2 = jnp.maximum(h2 + b2_ref[...], 0.0)
    h3 = jnp.dot(h2.astype(jnp.bfloat16), w3_ref[...],
                 preferred_element_type=jnp.float32)
    h3 = jnp.maximum(h3 + b3_ref[...], 0.0)

    outdim = w3_ref.shape[1] // n_exp
    outs = [jnp.zeros((tb, outdim), jnp.float32) for _ in range(n_task)]
    for e in range(n_exp):
        h3e = h3[:, e * outdim:(e + 1) * outdim]
        for t in range(n_task):
            outs[t] = outs[t] + gates[t][:, e:e + 1] * h3e

    for t in range(n_task):
        out_ref[t, :, :] = outs[t]

    imp = jnp.concatenate([jnp.sum(g, axis=0, keepdims=True) for g in gates],
                          axis=0)
    load = jnp.concatenate(
        [jnp.sum((g > 0.0).astype(jnp.float32), axis=0, keepdims=True)
         for g in gates], axis=0)
    upd = jnp.concatenate(
        [imp, load,
         jnp.zeros((8 - 2 * len(gates), imp.shape[1]), jnp.float32)], axis=0)

    @pl.when(pl.program_id(0) == 0)
    def _init():
        stats_ref[...] = jnp.zeros_like(stats_ref)

    stats_ref[...] += upd


def _cv_squared(v):
    eps = 1e-10
    return jnp.var(v, ddof=1) / (jnp.mean(v) ** 2 + eps)


@functools.partial(jax.jit, static_argnames=())
def kernel(x, W1, b1, W2, b2, W3, b3, wg, bg):
    B, D = x.shape
    E, _, H1 = W1.shape
    T = wg.shape[0]
    OUT = W3.shape[2]

    H2 = W2.shape[2]
    TB = 512 if B % 512 == 0 else B
    grid = (B // TB,)

    w1c = W1.transpose(1, 0, 2).reshape(D, E * H1).astype(jnp.bfloat16)
    b1c = b1.reshape(1, E * H1)
    w2bd = jax.scipy.linalg.block_diag(*[W2[e] for e in range(E)])
    w2bd = w2bd.astype(jnp.bfloat16)
    b2c = b2.reshape(1, E * H2)
    w3bd = jax.scipy.linalg.block_diag(*[W3[e] for e in range(E)])
    w3bd = w3bd.astype(jnp.bfloat16)
    b3c = b3.reshape(1, E * OUT)
    wgc = wg.transpose(1, 0, 2).reshape(D, T * E)
    bgc = bg.reshape(1, T * E)

    out, stats = pl.pallas_call(
        functools.partial(_moe_kernel, n_task=T, n_exp=E),
        grid=grid,
        in_specs=[
            pl.BlockSpec((TB, D), lambda i: (i, 0)),
            pl.BlockSpec((D, E * H1), lambda i: (0, 0)),
            pl.BlockSpec((1, E * H1), lambda i: (0, 0)),
            pl.BlockSpec((E * H1, E * H2), lambda i: (0, 0)),
            pl.BlockSpec((1, E * H2), lambda i: (0, 0)),
            pl.BlockSpec((E * H2, E * OUT), lambda i: (0, 0)),
            pl.BlockSpec((1, E * OUT), lambda i: (0, 0)),
            pl.BlockSpec((D, T * E), lambda i: (0, 0)),
            pl.BlockSpec((1, T * E), lambda i: (0, 0)),
        ],
        out_specs=[
            pl.BlockSpec((T, TB, OUT), lambda i: (0, i, 0)),
            pl.BlockSpec((8, E), lambda i: (0, 0)),
        ],
        out_shape=[
            jax.ShapeDtypeStruct((T, B, OUT), jnp.float32),
            jax.ShapeDtypeStruct((8, E), jnp.float32),
        ],
        compiler_params=pltpu.CompilerParams(
            dimension_semantics=("arbitrary",)),
    )(x, w1c, b1c, w2bd, b2c, w3bd, b3c, wgc, bgc)

    imp = stats[0:T, :]
    load = stats[T:2 * T, :]
    loss = jnp.float32(0.0)
    for t in range(T):
        loss = loss + (_cv_squared(imp[t]) + _cv_squared(load[t])) * 0.01
    return out, loss


# speedup vs baseline: 1.1568x; 1.1568x over previous
import functools

import jax
import jax.numpy as jnp
from jax.experimental import pallas as pl
from jax.experimental.pallas import tpu as pltpu


def _moe_kernel(x_ref, w1_ref, b1_ref, w2_ref, b2_ref, w3_ref, b3_ref,
                wg_ref, bg_ref, s_ref, out_ref, stats_ref, *, n_task, n_exp):
    xb = x_ref[...].astype(jnp.bfloat16)
    tb = xb.shape[0]

    h1 = jnp.dot(xb, w1_ref[...], preferred_element_type=jnp.float32)
    h1 = jnp.maximum(h1 + b1_ref[...], 0.0)

    logits = jnp.dot(xb, wg_ref[...], preferred_element_type=jnp.float32)
    logits = logits + bg_ref[...]

    cols = jax.lax.broadcasted_iota(jnp.int32, (tb, n_exp), 1)

    gates = []
    for t in range(n_task):
        lt = logits[:, t * n_exp:(t + 1) * n_exp]
        m1 = jnp.max(lt, axis=1, keepdims=True)
        i1 = jnp.min(jnp.where(lt == m1, cols, n_exp), axis=1, keepdims=True)
        sel1 = cols == i1
        masked = jnp.where(sel1, -jnp.inf, lt)
        m2 = jnp.max(masked, axis=1, keepdims=True)
        i2 = jnp.min(jnp.where(masked == m2, cols, n_exp), axis=1,
                     keepdims=True)
        sel2 = cols == i2
        z = jnp.exp(m2 - m1)
        g1 = 1.0 / (1.0 + z)
        g2 = z / (1.0 + z)
        gates.append(jnp.where(sel1, g1, 0.0) + jnp.where(sel2, g2, 0.0))

    h2 = jnp.dot(h1.astype(jnp.bfloat16), w2_ref[...],
                 preferred_element_type=jnp.float32)
    h2 = jnp.maximum(h2 + b2_ref[...], 0.0)
    h3 = jnp.dot(h2.astype(jnp.bfloat16), w3_ref[...],
                 preferred_element_type=jnp.float32)
    h3 = jnp.maximum(h3 + b3_ref[...], 0.0)

    outdim = w3_ref.shape[1] // n_exp
    for t in range(n_task):
        grep = jnp.dot(gates[t].astype(jnp.bfloat16), s_ref[...],
                       preferred_element_type=jnp.float32)
        prod = grep * h3
        acc = prod[:, 0:outdim]
        for e in range(1, n_exp):
            acc = acc + prod[:, e * outdim:(e + 1) * outdim]
        out_ref[t, :, :] = acc

    imp = jnp.concatenate([jnp.sum(g, axis=0, keepdims=True) for g in gates],
                          axis=0)
    load = jnp.concatenate(
        [jnp.sum((g > 0.0).astype(jnp.float32), axis=0, keepdims=True)
         for g in gates], axis=0)
    upd = jnp.concatenate(
        [imp, load,
         jnp.zeros((8 - 2 * len(gates), imp.shape[1]), jnp.float32)], axis=0)

    @pl.when(pl.program_id(0) == 0)
    def _init():
        stats_ref[...] = jnp.zeros_like(stats_ref)

    stats_ref[...] += upd


def _cv_squared(v):
    eps = 1e-10
    return jnp.var(v, ddof=1) / (jnp.mean(v) ** 2 + eps)


@functools.partial(jax.jit, static_argnames=())
def kernel(x, W1, b1, W2, b2, W3, b3, wg, bg):
    B, D = x.shape
    E, _, H1 = W1.shape
    T = wg.shape[0]
    OUT = W3.shape[2]
    H2 = W2.shape[2]
    TB = 1024 if B % 1024 == 0 else B
    grid = (B // TB,)

    w1c = W1.transpose(1, 0, 2).reshape(D, E * H1).astype(jnp.bfloat16)
    b1c = b1.reshape(1, E * H1)
    w2bd = jax.scipy.linalg.block_diag(*[W2[e] for e in range(E)])
    w2bd = w2bd.astype(jnp.bfloat16)
    b2c = b2.reshape(1, E * H2)
    w3bd = jax.scipy.linalg.block_diag(*[W3[e] for e in range(E)])
    w3bd = w3bd.astype(jnp.bfloat16)
    b3c = b3.reshape(1, E * OUT)
    wgc = wg.transpose(1, 0, 2).reshape(D, T * E).astype(jnp.bfloat16)
    bgc = bg.reshape(1, T * E)
    srep = jnp.kron(jnp.eye(E), jnp.ones((1, OUT))).astype(jnp.bfloat16)

    out, stats = pl.pallas_call(
        functools.partial(_moe_kernel, n_task=T, n_exp=E),
        grid=grid,
        in_specs=[
            pl.BlockSpec((TB, D), lambda i: (i, 0)),
            pl.BlockSpec((D, E * H1), lambda i: (0, 0)),
            pl.BlockSpec((1, E * H1), lambda i: (0, 0)),
            pl.BlockSpec((E * H1, E * H2), lambda i: (0, 0)),
            pl.BlockSpec((1, E * H2), lambda i: (0, 0)),
            pl.BlockSpec((E * H2, E * OUT), lambda i: (0, 0)),
            pl.BlockSpec((1, E * OUT), lambda i: (0, 0)),
            pl.BlockSpec((D, T * E), lambda i: (0, 0)),
            pl.BlockSpec((1, T * E), lambda i: (0, 0)),
            pl.BlockSpec((E, E * OUT), lambda i: (0, 0)),
        ],
        out_specs=[
            pl.BlockSpec((T, TB, OUT), lambda i: (0, i, 0)),
            pl.BlockSpec((8, E), lambda i: (0, 0)),
        ],
        out_shape=[
            jax.ShapeDtypeStruct((T, B, OUT), jnp.float32),
            jax.ShapeDtypeStruct((8, E), jnp.float32),
        ],
        compiler_params=pltpu.CompilerParams(
            dimension_semantics=("arbitrary",)),
    )(x, w1c, b1c, w2bd, b2c, w3bd, b3c, wgc, bgc, srep)

    imp = stats[0:T, :]
    load = stats[T:2 * T, :]
    loss = jnp.float32(0.0)
    for t in range(T):
        loss = loss + (_cv_squared(imp[t]) + _cv_squared(load[t])) * 0.01
    return out, loss


# speedup vs baseline: 1.3773x; 1.1907x over previous
import functools

import jax
import jax.numpy as jnp
from jax.experimental import pallas as pl
from jax.experimental.pallas import tpu as pltpu


def _moe_kernel(x_ref, w1_ref, b1_ref, w2_ref, b2_ref, w3_ref, b3_ref,
                bg_ref, s_ref, out_ref, stats_ref, *, n_task, n_exp):
    xb = x_ref[...].astype(jnp.bfloat16)
    tb = xb.shape[0]
    nh1 = w1_ref.shape[1] - n_task * n_exp

    h1l = jnp.dot(xb, w1_ref[...], preferred_element_type=jnp.float32)
    h1 = jnp.maximum(h1l[:, :nh1] + b1_ref[...], 0.0)
    logits = h1l[:, nh1:] + bg_ref[...]

    gates = []
    for t in range(n_task):
        lt = logits[:, t * n_exp:(t + 1) * n_exp]
        m1 = jnp.max(lt, axis=1, keepdims=True)
        sel1 = lt == m1
        masked = jnp.where(sel1, -jnp.inf, lt)
        m2 = jnp.max(masked, axis=1, keepdims=True)
        sel2 = masked == m2
        z = jnp.exp(m2 - m1)
        g1 = 1.0 / (1.0 + z)
        g2 = z / (1.0 + z)
        gates.append(jnp.where(sel1, g1, 0.0) + jnp.where(sel2, g2, 0.0))

    h2 = jnp.dot(h1.astype(jnp.bfloat16), w2_ref[...],
                 preferred_element_type=jnp.float32)
    h2 = jnp.maximum(h2 + b2_ref[...], 0.0)
    h3 = jnp.dot(h2.astype(jnp.bfloat16), w3_ref[...],
                 preferred_element_type=jnp.float32)
    h3 = jnp.maximum(h3 + b3_ref[...], 0.0)

    outdim = w3_ref.shape[1] // n_exp
    for t in range(n_task):
        grep = jnp.dot(gates[t].astype(jnp.bfloat16), s_ref[...],
                       preferred_element_type=jnp.float32)
        prod = grep * h3
        acc = prod[:, 0:outdim]
        for e in range(1, n_exp):
            acc = acc + prod[:, e * outdim:(e + 1) * outdim]
        out_ref[t, :, :] = acc

    imp = jnp.concatenate([jnp.sum(g, axis=0, keepdims=True) for g in gates],
                          axis=0)
    load = jnp.concatenate(
        [jnp.sum((g > 0.0).astype(jnp.float32), axis=0, keepdims=True)
         for g in gates], axis=0)
    upd = jnp.concatenate(
        [imp, load,
         jnp.zeros((8 - 2 * len(gates), imp.shape[1]), jnp.float32)], axis=0)

    @pl.when(pl.program_id(0) == 0)
    def _init():
        stats_ref[...] = jnp.zeros_like(stats_ref)

    stats_ref[...] += upd


def _cv_squared(v):
    eps = 1e-10
    return jnp.var(v, ddof=1) / (jnp.mean(v) ** 2 + eps)


@functools.partial(jax.jit, static_argnames=())
def kernel(x, W1, b1, W2, b2, W3, b3, wg, bg):
    B, D = x.shape
    E, _, H1 = W1.shape
    T = wg.shape[0]
    OUT = W3.shape[2]
    H2 = W2.shape[2]
    TB = 1024 if B % 1024 == 0 else B
    grid = (B // TB,)

    w1c = jnp.concatenate(
        [W1.transpose(1, 0, 2).reshape(D, E * H1),
         wg.transpose(1, 0, 2).reshape(D, T * E)], axis=1).astype(jnp.bfloat16)
    b1c = b1.reshape(1, E * H1)
    w2bd = jax.scipy.linalg.block_diag(*[W2[e] for e in range(E)])
    w2bd = w2bd.astype(jnp.bfloat16)
    b2c = b2.reshape(1, E * H2)
    w3bd = jax.scipy.linalg.block_diag(*[W3[e] for e in range(E)])
    w3bd = w3bd.astype(jnp.bfloat16)
    b3c = b3.reshape(1, E * OUT)
    bgc = bg.reshape(1, T * E)
    srep = jnp.kron(jnp.eye(E), jnp.ones((1, OUT))).astype(jnp.bfloat16)

    out, stats = pl.pallas_call(
        functools.partial(_moe_kernel, n_task=T, n_exp=E),
        grid=grid,
        in_specs=[
            pl.BlockSpec((TB, D), lambda i: (i, 0)),
            pl.BlockSpec((D, E * H1 + T * E), lambda i: (0, 0)),
            pl.BlockSpec((1, E * H1), lambda i: (0, 0)),
            pl.BlockSpec((E * H1, E * H2), lambda i: (0, 0)),
            pl.BlockSpec((1, E * H2), lambda i: (0, 0)),
            pl.BlockSpec((E * H2, E * OUT), lambda i: (0, 0)),
            pl.BlockSpec((1, E * OUT), lambda i: (0, 0)),
            pl.BlockSpec((1, T * E), lambda i: (0, 0)),
            pl.BlockSpec((E, E * OUT), lambda i: (0, 0)),
        ],
        out_specs=[
            pl.BlockSpec((T, TB, OUT), lambda i: (0, i, 0)),
            pl.BlockSpec((8, E), lambda i: (0, 0)),
        ],
        out_shape=[
            jax.ShapeDtypeStruct((T, B, OUT), jnp.float32),
            jax.ShapeDtypeStruct((8, E), jnp.float32),
        ],
        compiler_params=pltpu.CompilerParams(
            dimension_semantics=("arbitrary",)),
    )(x, w1c, b1c, w2bd, b2c, w3bd, b3c, bgc, srep)

    imp = stats[0:T, :]
    load = stats[T:2 * T, :]
    loss = jnp.float32(0.0)
    for t in range(T):
        loss = loss + (_cv_squared(imp[t]) + _cv_squared(load[t])) * 0.01
    return out, loss
